# baseline (speedup 1.0000x reference)
import functools

import jax
import jax.numpy as jnp
from jax import lax
from jax.experimental import pallas as pl
from jax.experimental.pallas import tpu as pltpu
from jax.experimental.pallas import tpu_sc as plsc


def _gate_pool_kernel(x_ref, ids_ref, gW1_ref, gb1_ref, gW2_ref, gb2_ref,
                      e_ref, praw_ref):
    x = x_ref[0]
    h = jnp.tanh(jnp.dot(x, gW1_ref[...],
                         preferred_element_type=jnp.float32) + gb1_ref[...])
    g_col = jnp.dot(h, gW2_ref[...],
                    preferred_element_type=jnp.float32)
    g_row = g_col.T + gb2_ref[...]
    e_row = jnp.exp(g_row)
    e_ref[0] = e_row
    n = x.shape[0]
    c = praw_ref.shape[1]
    onehot = lax.broadcasted_iota(jnp.int32, (c, n), 0) == ids_ref[0]
    numer = jnp.where(onehot, e_row, 0.0)
    praw_ref[0] = jnp.dot(numer, x,
                          preferred_element_type=jnp.float32)


def _make_seg_sum(BN, C):
    NC, NS, L = 2, 16, 16
    NW = NC * NS
    P = BN // NW
    CP = ((C + 1 + L - 1) // L) * L
    CO = (C // L) * L
    mesh = plsc.VectorSubcoreMesh(core_axis_name="c", subcore_axis_name="s")

    @functools.partial(
        pl.kernel,
        mesh=mesh,
        out_type=jax.ShapeDtypeStruct((NW, C), jnp.float32),
        compiler_params=pltpu.CompilerParams(needs_layout_passes=False),
        scratch_types=[
            pltpu.VMEM((P,), jnp.float32),
            pltpu.VMEM((P,), jnp.int32),
            pltpu.VMEM((L * CP,), jnp.float32),
            pltpu.VMEM((C,), jnp.float32),
        ],
    )
    def seg_sum(e_hbm, ids_hbm, out_hbm, e_v, ids_v, acc_v, s_v):
        wid = lax.axis_index("s") * NC + lax.axis_index("c")
        base = wid * P
        pltpu.sync_copy(e_hbm.at[pl.ds(base, P)], e_v)
        pltpu.sync_copy(ids_hbm.at[pl.ds(base, P)], ids_v)
        zeros = jnp.zeros((L,), jnp.float32)
        for r in range(L * CP // L):
            acc_v[pl.ds(r * L, L)] = zeros
        lane_off = lax.iota(jnp.int32, L) * CP
        for i in range(P // L):
            ev = e_v[pl.ds(i * L, L)]
            iv = ids_v[pl.ds(i * L, L)] + lane_off
            av = plsc.load_gather(acc_v, [iv])
            plsc.store_scatter(acc_v, [iv], av + ev)
        for j in range(CO // L):
            sv = acc_v[pl.ds(j * L, L)]
            for r in range(1, L):
                sv = sv + acc_v[pl.ds(r * CP + j * L, L)]
            s_v[pl.ds(j * L, L)] = sv
        pltpu.sync_copy(s_v, out_hbm.at[wid])

    return seg_sum, NW


def _proj_kernel(praw_ref, st_ref, embT_ref, ct_ref, pW_ref, pb_ref,
                 lng_ref, lnb_ref, out_ref):
    b = pl.program_id(0)
    c, nb = st_ref.shape
    t = embT_ref.shape[1]
    onehot_b = (lax.broadcasted_iota(jnp.int32, (nb, 1), 0) == b
                ).astype(jnp.float32)
    s_col = jnp.dot(st_ref[...], onehot_b,
                    preferred_element_type=jnp.float32)
    ct_b = ct_ref[b]
    onehot_t = (lax.broadcasted_iota(jnp.int32, (t, 1), 0) == ct_b
                ).astype(jnp.float32)
    ctm_col = jnp.dot(embT_ref[...], onehot_t,
                      preferred_element_type=jnp.float32)
    any_col = jnp.sum(st_ref[...], axis=1, keepdims=True) > 0.0
    nonempty = s_col > 0.0
    scale = jnp.where(nonempty,
                      (1.0 + 0.1 * ctm_col) / jnp.where(nonempty, s_col, 1.0),
                      0.0)
    pooled = praw_ref[0] * scale
    proj = jnp.dot(pooled, pW_ref[...],
                   preferred_element_type=jnp.float32) + pb_ref[...]
    mean = jnp.mean(proj, axis=1, keepdims=True)
    d = proj - mean
    var = jnp.mean(d * d, axis=1, keepdims=True)
    y = d * lax.rsqrt(var + 1e-5) * lng_ref[...] + lnb_ref[...]
    y = jnp.where(y > 0.0, y, jnp.exp(jnp.minimum(y, 0.0)) - 1.0)
    out_ref[0] = jnp.where(any_col, y, 0.0)


def kernel(x, gW1, gb1, gW2, gb2, emb, pW, pb, ln_g, ln_b, cancer_type,
           channel_ids, pad_mask):
    B, N, H = x.shape
    T, C = emb.shape
    ids_m = jnp.where(pad_mask, C, channel_ids.astype(jnp.int32))
    ids2 = ids_m.reshape(B, 1, N)

    e, praw = pl.pallas_call(
        _gate_pool_kernel,
        grid=(B,),
        in_specs=[
            pl.BlockSpec((1, N, H), lambda b: (b, 0, 0)),
            pl.BlockSpec((1, 1, N), lambda b: (b, 0, 0)),
            pl.BlockSpec((H, H // 2), lambda b: (0, 0)),
            pl.BlockSpec((1, H // 2), lambda b: (0, 0)),
            pl.BlockSpec((H // 2, 1), lambda b: (0, 0)),
            pl.BlockSpec((1, 1), lambda b: (0, 0)),
        ],
        out_specs=[
            pl.BlockSpec((1, 1, N), lambda b: (b, 0, 0)),
            pl.BlockSpec((1, C, H), lambda b: (b, 0, 0)),
        ],
        out_shape=[
            jax.ShapeDtypeStruct((B, 1, N), jnp.float32),
            jax.ShapeDtypeStruct((B, C, H), jnp.float32),
        ],
    )(x, ids2, gW1, gb1.reshape(1, -1), gW2, gb2.reshape(1, 1))

    seg_sum, NW = _make_seg_sum(B * N, C)
    partials = seg_sum(e.reshape(B * N), ids_m.reshape(B * N))
    Smat = partials.reshape(B, NW // B, C).sum(axis=1)

    tokens = pl.pallas_call(
        _proj_kernel,
        grid=(B,),
        in_specs=[
            pl.BlockSpec((1, C, H), lambda b: (b, 0, 0)),
            pl.BlockSpec((C, B), lambda b: (0, 0)),
            pl.BlockSpec((C, T), lambda b: (0, 0)),
            pl.BlockSpec(memory_space=pltpu.SMEM),
            pl.BlockSpec((H, H), lambda b: (0, 0)),
            pl.BlockSpec((1, H), lambda b: (0, 0)),
            pl.BlockSpec((1, H), lambda b: (0, 0)),
            pl.BlockSpec((1, H), lambda b: (0, 0)),
        ],
        out_specs=pl.BlockSpec((1, C, H), lambda b: (b, 0, 0)),
        out_shape=jax.ShapeDtypeStruct((B, C, H), jnp.float32),
    )(praw, Smat.T, emb.T, cancer_type.astype(jnp.int32), pW,
      pb.reshape(1, -1), ln_g.reshape(1, -1), ln_b.reshape(1, -1))

    channel_active = Smat > 0.0
    return tokens, channel_active


# speedup vs baseline: 1.0886x; 1.0886x over previous
import functools

import jax
import jax.numpy as jnp
from jax import lax
from jax.experimental import pallas as pl
from jax.experimental.pallas import tpu as pltpu
from jax.experimental.pallas import tpu_sc as plsc


def _gate_pool_kernel(x_ref, ids_ref, pad_ref, gW1_ref, gb1_ref, gW2_ref,
                      gb2_ref, s_ref, praw_ref):
    x = x_ref[0]
    h = jnp.tanh(jnp.dot(x, gW1_ref[...],
                         preferred_element_type=jnp.float32) + gb1_ref[...])
    g_col = jnp.dot(h, gW2_ref[...],
                    preferred_element_type=jnp.float32)
    g_row = g_col.T + gb2_ref[...]
    e_row = jnp.where(pad_ref[0] != 0, 0.0, jnp.exp(g_row))
    n = x.shape[0]
    c = praw_ref.shape[1]
    onehot = lax.broadcasted_iota(jnp.int32, (c, n), 0) == ids_ref[0]
    numer = jnp.where(onehot, e_row, 0.0)
    s_ref[0] = jnp.sum(numer, axis=1, keepdims=True)
    praw_ref[0] = jnp.dot(numer, x,
                          preferred_element_type=jnp.float32)


def _make_seg_count(BN, C):
    NC, NS, L = 2, 16, 16
    NW = NC * NS
    P = BN // NW
    mesh = plsc.VectorSubcoreMesh(core_axis_name="c", subcore_axis_name="s")

    @functools.partial(
        pl.kernel,
        mesh=mesh,
        out_type=jax.ShapeDtypeStruct((NW, C), jnp.float32),
        compiler_params=pltpu.CompilerParams(needs_layout_passes=False),
        scratch_types=[
            pltpu.VMEM((P,), jnp.int32),
            pltpu.VMEM((P,), jnp.int32),
            pltpu.VMEM((L * C,), jnp.float32),
            pltpu.VMEM((C,), jnp.float32),
        ],
    )
    def seg_count(ids_hbm, pad_hbm, out_hbm, ids_v, pad_v, acc_v, s_v):
        wid = lax.axis_index("s") * NC + lax.axis_index("c")
        base = wid * P
        pltpu.sync_copy(ids_hbm.at[pl.ds(base, P)], ids_v)
        pltpu.sync_copy(pad_hbm.at[pl.ds(base, P)], pad_v)
        zeros = jnp.zeros((L,), jnp.float32)
        for r in range(L * C // L):
            acc_v[pl.ds(r * L, L)] = zeros
        lane_off = lax.iota(jnp.int32, L) * C
        one = jnp.ones((L,), jnp.float32)
        for i in range(P // L):
            nv = one - pad_v[pl.ds(i * L, L)].astype(jnp.float32)
            iv = ids_v[pl.ds(i * L, L)] + lane_off
            av = plsc.load_gather(acc_v, [iv])
            plsc.store_scatter(acc_v, [iv], av + nv)
        for j in range(C // L):
            sv = acc_v[pl.ds(j * L, L)]
            for r in range(1, L):
                sv = sv + acc_v[pl.ds(r * C + j * L, L)]
            s_v[pl.ds(j * L, L)] = sv
        pltpu.sync_copy(s_v, out_hbm.at[wid])

    return seg_count, NW


def _proj_kernel(praw_ref, st_ref, embT_ref, ct_ref, pW_ref, pb_ref,
                 lng_ref, lnb_ref, out_ref):
    c, nb = st_ref.shape
    t = embT_ref.shape[1]
    onehot_tb = (lax.broadcasted_iota(jnp.int32, (t, nb), 0) == ct_ref[...]
                 ).astype(jnp.float32)
    ctmT = jnp.dot(embT_ref[...], onehot_tb,
                   preferred_element_type=jnp.float32)
    st = st_ref[...]
    ne = st > 0.0
    scaleT = jnp.where(ne, (1.0 + 0.1 * ctmT) / jnp.where(ne, st, 1.0), 0.0)
    any_col = jnp.sum(st, axis=1, keepdims=True) > 0.0
    pW = pW_ref[...]
    pb = pb_ref[...]
    lng = lng_ref[...]
    lnb = lnb_ref[...]
    for b in range(nb):
        pooled = praw_ref[b] * scaleT[:, b:b + 1]
        proj = jnp.dot(pooled, pW, preferred_element_type=jnp.float32) + pb
        mean = jnp.mean(proj, axis=1, keepdims=True)
        d = proj - mean
        var = jnp.mean(d * d, axis=1, keepdims=True)
        y = d * lax.rsqrt(var + 1e-5) * lng + lnb
        y = jnp.where(y > 0.0, y, jnp.exp(jnp.minimum(y, 0.0)) - 1.0)
        out_ref[b] = jnp.where(any_col, y, 0.0)


def kernel(x, gW1, gb1, gW2, gb2, emb, pW, pb, ln_g, ln_b, cancer_type,
           channel_ids, pad_mask):
    B, N, H = x.shape
    T, C = emb.shape
    ids_i = channel_ids.astype(jnp.int32)
    pad_i = pad_mask.astype(jnp.int32)

    seg_count, NW = _make_seg_count(B * N, C)
    counts = seg_count(ids_i.reshape(B * N), pad_i.reshape(B * N))
    channel_active = counts.reshape(B, NW // B, C).sum(axis=1) > 0.0

    S, praw = pl.pallas_call(
        _gate_pool_kernel,
        grid=(B,),
        in_specs=[
            pl.BlockSpec((1, N, H), lambda b: (b, 0, 0)),
            pl.BlockSpec((1, 1, N), lambda b: (b, 0, 0)),
            pl.BlockSpec((1, 1, N), lambda b: (b, 0, 0)),
            pl.BlockSpec((H, H // 2), lambda b: (0, 0)),
            pl.BlockSpec((1, H // 2), lambda b: (0, 0)),
            pl.BlockSpec((H // 2, 1), lambda b: (0, 0)),
            pl.BlockSpec((1, 1), lambda b: (0, 0)),
        ],
        out_specs=[
            pl.BlockSpec((1, C, 1), lambda b: (b, 0, 0)),
            pl.BlockSpec((1, C, H), lambda b: (b, 0, 0)),
        ],
        out_shape=[
            jax.ShapeDtypeStruct((B, C, 1), jnp.float32),
            jax.ShapeDtypeStruct((B, C, H), jnp.float32),
        ],
    )(x, ids_i.reshape(B, 1, N), pad_i.reshape(B, 1, N), gW1,
      gb1.reshape(1, -1), gW2, gb2.reshape(1, 1))

    tokens = pl.pallas_call(
        _proj_kernel,
        grid=(1,),
        in_specs=[
            pl.BlockSpec((B, C, H), lambda i: (0, 0, 0)),
            pl.BlockSpec((C, B), lambda i: (0, 0)),
            pl.BlockSpec((C, T), lambda i: (0, 0)),
            pl.BlockSpec((1, B), lambda i: (0, 0)),
            pl.BlockSpec((H, H), lambda i: (0, 0)),
            pl.BlockSpec((1, H), lambda i: (0, 0)),
            pl.BlockSpec((1, H), lambda i: (0, 0)),
            pl.BlockSpec((1, H), lambda i: (0, 0)),
        ],
        out_specs=pl.BlockSpec((B, C, H), lambda i: (0, 0, 0)),
        out_shape=jax.ShapeDtypeStruct((B, C, H), jnp.float32),
    )(praw, S[:, :, 0].T, emb.T, cancer_type.astype(jnp.int32).reshape(1, B),
      pW, pb.reshape(1, -1), ln_g.reshape(1, -1), ln_b.reshape(1, -1))

    return tokens, channel_active
